# baseline (speedup 1.0000x reference)
import jax
import jax.numpy as jnp
from jax import lax
from jax.experimental import pallas as pl
from jax.experimental.pallas import tpu as pltpu
from jax.experimental.pallas import tpu_sc as plsc

NUM_FACTORS = 32
LANES = 16
CHUNK = 128


def kernel(user_id, item_id, user_emb, item_emb, user_bias, item_bias):
    batch = user_id.shape[0]
    info = plsc.get_sparse_core_info()
    nc, ns = info.num_cores, info.num_subcores
    nw = nc * ns
    b_per_w = batch // nw

    def body(uid_hbm, iid_hbm, uemb_hbm, iemb_hbm, ubias_hbm, ibias_hbm,
             out_hbm, uid_v, iid_v, uq_v, iq_v, urows, irows, ubrows, ibrows,
             out_v, sem):
        wid = lax.axis_index("s") * nc + lax.axis_index("c")
        base = wid * b_per_w
        pltpu.sync_copy(uid_hbm.at[pl.ds(base, b_per_w)], uid_v)
        pltpu.sync_copy(iid_hbm.at[pl.ds(base, b_per_w)], iid_v)

        def idx_body(k, carry):
            sl = pl.ds(k * LANES, LANES)
            uq_v[sl] = lax.shift_right_logical(uid_v[sl], 4)
            iq_v[sl] = lax.shift_right_logical(iid_v[sl], 4)
            return carry

        lax.fori_loop(0, b_per_w // LANES, idx_body, 0)

        copies = []
        for c in range(b_per_w // CHUNK):
            sl = pl.ds(c * CHUNK, CHUNK)
            copies.append(pltpu.async_copy(uemb_hbm.at[uid_v.at[sl]], urows.at[sl], sem))
            copies.append(pltpu.async_copy(iemb_hbm.at[iid_v.at[sl]], irows.at[sl], sem))
            copies.append(pltpu.async_copy(ubias_hbm.at[uq_v.at[sl]], ubrows.at[sl], sem))
            copies.append(pltpu.async_copy(ibias_hbm.at[iq_v.at[sl]], ibrows.at[sl], sem))
        for cp in copies:
            cp.wait()

        lane = lax.iota(jnp.int32, LANES)
        zero = jnp.zeros((LANES,), jnp.float32)

        def blk_body(blk, carry):
            sl = pl.ds(blk * LANES, LANES)
            uqs = jnp.bitwise_and(uid_v[sl], 15)
            iqs = jnp.bitwise_and(iid_v[sl], 15)
            res = zero
            for r in range(LANES):
                row = blk * LANES + r
                v = (urows[row, pl.ds(0, LANES)] * irows[row, pl.ds(0, LANES)] +
                     urows[row, pl.ds(LANES, LANES)] * irows[row, pl.ds(LANES, LANES)])
                v = v + jnp.where(lane == uqs[r], ubrows[row, pl.ds(0, LANES)], zero)
                v = v + jnp.where(lane == iqs[r], ibrows[row, pl.ds(0, LANES)], zero)
                for k in (1, 2, 4, 8):
                    v = v + v.at[lane ^ k].get(mode="promise_in_bounds")
                res = jnp.where(lane == r, v, res)
            out_v[sl] = res
            return carry

        lax.fori_loop(0, b_per_w // LANES, blk_body, 0)
        pltpu.sync_copy(out_v, out_hbm.at[pl.ds(base, b_per_w)])

    run = pl.kernel(
        body,
        out_type=jax.ShapeDtypeStruct((batch,), jnp.float32),
        mesh=plsc.VectorSubcoreMesh(core_axis_name="c", subcore_axis_name="s"),
        compiler_params=pltpu.CompilerParams(use_tc_tiling_on_sc=False),
        scratch_types=[
            pltpu.VMEM((b_per_w,), jnp.int32),
            pltpu.VMEM((b_per_w,), jnp.int32),
            pltpu.VMEM((b_per_w,), jnp.int32),
            pltpu.VMEM((b_per_w,), jnp.int32),
            pltpu.VMEM((b_per_w, NUM_FACTORS), jnp.float32),
            pltpu.VMEM((b_per_w, NUM_FACTORS), jnp.float32),
            pltpu.VMEM((b_per_w, LANES), jnp.float32),
            pltpu.VMEM((b_per_w, LANES), jnp.float32),
            pltpu.VMEM((b_per_w,), jnp.float32),
            pltpu.SemaphoreType.DMA,
        ],
    )
    return run(user_id.astype(jnp.int32), item_id.astype(jnp.int32),
               user_emb, item_emb,
               user_bias.reshape(-1, LANES), item_bias.reshape(-1, LANES))
